# baseline (speedup 1.0000x reference)
import functools

import jax
import jax.numpy as jnp
from jax import lax
from jax.experimental import pallas as pl
from jax.experimental.pallas import tpu as pltpu
from jax.experimental.pallas import tpu_sc as plsc

_BLK = 20000
_NUM_GRAPHS = 64
_SHIFT = 0.6931471805599453

_N = 100000
_OUT = 128
_Z_CH = 400
_Z_NCH = _N // _Z_CH
_Z_NW = 32


def _tc_body(nblk, v_ref, b_ref, w1_ref, b1_ref, w2_ref, b2_ref, out_ref,
             acc_ref, cnt_ref):
    i = pl.program_id(0)

    @pl.when(i == 0)
    def _init():
        acc_ref[...] = jnp.zeros_like(acc_ref)
        cnt_ref[...] = jnp.zeros_like(cnt_ref)

    x = v_ref[...]
    y = jnp.dot(x, w1_ref[...], preferred_element_type=jnp.float32)
    y = y + b1_ref[...]
    h = (jnp.maximum(y, 0.0) - 1.0) + jnp.log2(1.0 + jnp.exp2(jnp.minimum(y, -y)))

    seg = b_ref[0]
    gids = lax.broadcasted_iota(jnp.int32, (_NUM_GRAPHS, seg.shape[1]), 0)
    oh = (gids == seg).astype(jnp.float32)
    acc_ref[...] += jnp.dot(oh, h, preferred_element_type=jnp.float32)
    cnt_ref[...] += jnp.sum(oh, axis=1, keepdims=True)

    @pl.when(i == nblk - 1)
    def _finish():
        u0 = jnp.dot(acc_ref[...], w2_ref[...],
                     preferred_element_type=jnp.float32)
        out_ref[...] = u0 + cnt_ref[...] * b2_ref[...]


def _sc_zero_body(out_hbm, zb):
    wid = lax.axis_index("s") * 2 + lax.axis_index("c")
    zb[...] = jnp.zeros_like(zb)
    for k in range(-(-_Z_NCH // _Z_NW)):
        ci = wid + k * _Z_NW

        @pl.when(ci < _Z_NCH)
        def _copy(ci=ci):
            pltpu.sync_copy(zb, out_hbm.at[pl.ds(ci * _Z_CH, _Z_CH)])


_sc_zeros = functools.partial(
    pl.kernel,
    out_type=jax.ShapeDtypeStruct((_N, _OUT), jnp.float32),
    mesh=plsc.VectorSubcoreMesh(core_axis_name="c", subcore_axis_name="s"),
    scratch_types=[pltpu.VMEM((_Z_CH, _OUT), jnp.float32)],
)(_sc_zero_body)


def kernel(v, batch, W1, b1, W2, b2):
    n, hidden = v.shape
    out_dim = W2.shape[1]
    nblk = n // _BLK
    batch_r = batch.astype(jnp.int32).reshape(nblk, 1, _BLK)
    log2e = 1.4426950408889634
    W1 = W1 * log2e
    b1r = (b1 * log2e).reshape(1, -1)
    W2 = W2 * _SHIFT
    b2r = b2.reshape(1, -1)
    u0 = pl.pallas_call(
        functools.partial(_tc_body, nblk),
        grid=(nblk,),
        in_specs=[
            pl.BlockSpec((_BLK, hidden), lambda i: (i, 0)),
            pl.BlockSpec((1, 1, _BLK), lambda i: (i, 0, 0)),
            pl.BlockSpec(W1.shape, lambda i: (0, 0)),
            pl.BlockSpec(b1r.shape, lambda i: (0, 0)),
            pl.BlockSpec(W2.shape, lambda i: (0, 0)),
            pl.BlockSpec(b2r.shape, lambda i: (0, 0)),
        ],
        out_specs=pl.BlockSpec((_NUM_GRAPHS, out_dim), lambda i: (0, 0)),
        out_shape=jax.ShapeDtypeStruct((_NUM_GRAPHS, out_dim), jnp.float32),
        scratch_shapes=[
            pltpu.VMEM((_NUM_GRAPHS, W1.shape[1]), jnp.float32),
            pltpu.VMEM((_NUM_GRAPHS, 1), jnp.float32),
        ],
    )(v, batch_r, W1, b1r, W2, b2r)
    z = _sc_zeros()
    return lax.dynamic_update_slice(z, u0, (0, 0))


# speedup vs baseline: 1.2736x; 1.2736x over previous
import functools

import jax
import jax.numpy as jnp
from jax import lax
from jax.experimental import pallas as pl
from jax.experimental.pallas import tpu as pltpu

_BLK = 20000
_NUM_GRAPHS = 64
_LN2 = 0.6931471805599453
_LOG2E = 1.4426950408889634


def _body(nblk, v_ref, b_ref, w1_ref, b1_ref, w2_ref, b2_ref, out_ref,
          acc_ref, cnt_ref):
    i = pl.program_id(0)

    @pl.when(i == 0)
    def _init():
        acc_ref[...] = jnp.zeros_like(acc_ref)
        cnt_ref[...] = jnp.zeros_like(cnt_ref)

    x = v_ref[...]
    y = jnp.dot(x, w1_ref[...] * _LOG2E, preferred_element_type=jnp.float32)
    y = y + b1_ref[...] * _LOG2E
    h = (jnp.maximum(y, 0.0) - 1.0) + jnp.log2(1.0 + jnp.exp2(jnp.minimum(y, -y)))

    seg = b_ref[0]
    gids = lax.broadcasted_iota(jnp.int32, (_NUM_GRAPHS, _BLK), 0)
    oh = (gids == seg).astype(jnp.float32)
    acc_ref[...] += jnp.dot(oh, h, preferred_element_type=jnp.float32)
    cnt_ref[...] += jnp.sum(oh, axis=1, keepdims=True)

    out_ref[...] = jnp.zeros_like(out_ref)

    @pl.when(i == nblk - 1)
    def _finish():
        u0 = jnp.dot(acc_ref[...], w2_ref[...],
                     preferred_element_type=jnp.float32) * _LN2
        out_ref[0:_NUM_GRAPHS, :] = u0 + cnt_ref[...] * b2_ref[...]


def kernel(v, batch, W1, b1, W2, b2):
    n, hidden = v.shape
    out_dim = W2.shape[1]
    nblk = n // _BLK
    batch_r = batch.astype(jnp.int32).reshape(nblk, 1, _BLK)
    b1r = b1.reshape(1, -1)
    b2r = b2.reshape(1, -1)
    return pl.pallas_call(
        functools.partial(_body, nblk),
        grid=(nblk,),
        in_specs=[
            pl.BlockSpec((_BLK, hidden), lambda i: (i, 0)),
            pl.BlockSpec((1, 1, _BLK), lambda i: (i, 0, 0)),
            pl.BlockSpec(W1.shape, lambda i: (0, 0)),
            pl.BlockSpec(b1r.shape, lambda i: (0, 0)),
            pl.BlockSpec(W2.shape, lambda i: (0, 0)),
            pl.BlockSpec(b2r.shape, lambda i: (0, 0)),
        ],
        out_specs=pl.BlockSpec((_BLK, out_dim), lambda i: ((i + 1) % nblk, 0)),
        out_shape=jax.ShapeDtypeStruct((n, out_dim), jnp.float32),
        scratch_shapes=[
            pltpu.VMEM((_NUM_GRAPHS, W1.shape[1]), jnp.float32),
            pltpu.VMEM((_NUM_GRAPHS, 1), jnp.float32),
        ],
    )(v, batch_r, W1, b1r, W2, b2r)


# speedup vs baseline: 1.3209x; 1.0371x over previous
import functools

import jax
import jax.numpy as jnp
from jax import lax
from jax.experimental import pallas as pl
from jax.experimental.pallas import tpu as pltpu

_BLK = 20000
_NUM_GRAPHS = 64
_LN2 = 0.6931471805599453
_LOG2E = 1.4426950408889634


def _body(nblk, v_ref, b_ref, w1_ref, b1_ref, w2_ref, b2_ref, out_ref,
          acc_ref, cnt_ref):
    i = pl.program_id(0)

    @pl.when(i == 0)
    def _init():
        acc_ref[...] = jnp.zeros_like(acc_ref)
        cnt_ref[...] = jnp.zeros_like(cnt_ref)

    x = v_ref[...]
    y = lax.dot_general(x, w1_ref[...] * _LOG2E, (((1,), (1,)), ((), ())),
                        preferred_element_type=jnp.float32)
    y = y + b1_ref[...] * _LOG2E
    h = (jnp.maximum(y, 0.0) - 1.0) + jnp.log2(1.0 + jnp.exp2(jnp.minimum(y, -y)))

    seg = b_ref[0]
    gids = lax.broadcasted_iota(jnp.int32, (_NUM_GRAPHS, _BLK), 0)
    oh = (gids == seg).astype(jnp.float32)
    acc_ref[...] += jnp.dot(oh, h, preferred_element_type=jnp.float32)
    cnt_ref[...] += jnp.sum(oh, axis=1, keepdims=True)

    out_ref[...] = jnp.zeros_like(out_ref)

    @pl.when(i == nblk - 1)
    def _finish():
        u0 = jnp.dot(acc_ref[...], w2_ref[...],
                     preferred_element_type=jnp.float32) * _LN2
        out_ref[0:_NUM_GRAPHS, :] = u0 + cnt_ref[...] * b2_ref[...]


def kernel(v, batch, W1, b1, W2, b2):
    n, hidden = v.shape
    out_dim = W2.shape[1]
    nblk = n // _BLK
    batch_r = batch.astype(jnp.int32).reshape(nblk, 1, _BLK)
    W1t = W1.T
    b1r = b1.reshape(1, -1)
    b2r = b2.reshape(1, -1)
    return pl.pallas_call(
        functools.partial(_body, nblk),
        grid=(nblk,),
        in_specs=[
            pl.BlockSpec((_BLK, hidden), lambda i: (i, 0)),
            pl.BlockSpec((1, 1, _BLK), lambda i: (i, 0, 0)),
            pl.BlockSpec(W1t.shape, lambda i: (0, 0)),
            pl.BlockSpec(b1r.shape, lambda i: (0, 0)),
            pl.BlockSpec(W2.shape, lambda i: (0, 0)),
            pl.BlockSpec(b2r.shape, lambda i: (0, 0)),
        ],
        out_specs=pl.BlockSpec((_BLK, out_dim), lambda i: ((i + 1) % nblk, 0)),
        out_shape=jax.ShapeDtypeStruct((n, out_dim), jnp.float32),
        scratch_shapes=[
            pltpu.VMEM((_NUM_GRAPHS, W1.shape[1]), jnp.float32),
            pltpu.VMEM((_NUM_GRAPHS, 1), jnp.float32),
        ],
    )(v, batch_r, W1t, b1r, W2, b2r)
